# Initial kernel scaffold; baseline (speedup 1.0000x reference)
import jax
import jax.numpy as jnp
from jax.experimental import pallas as pl


def kernel(x, table):
    raise NotImplementedError("write your pallas kernel here")


# speedup vs baseline: 1.0967x; 1.0967x over previous
import functools

import jax
import jax.numpy as jnp
from jax import lax
from jax.experimental import pallas as pl
from jax.experimental.pallas import tpu as pltpu
from jax.experimental.pallas import tpu_sc as plsc

_D = 300
_DP = 384
_N = 4096 * 50
_NC = 2
_NS = 16
_NW = _NC * _NS
_BPW = _N // _NW
_C = 128
_CHUNKS = _BPW // _C
_NBUF = 2


def _embed_body(idx_hbm, table_hbm, out_hbm, idx_v, rows_v, sem0, sem1):
    wid = lax.axis_index("s") * _NC + lax.axis_index("c")
    base = wid * _BPW
    sems = (sem0, sem1)

    def fire(b, c):
        pltpu.sync_copy(idx_hbm.at[pl.ds(base + c * _C, _C)], idx_v.at[b])
        pltpu.make_async_copy(
            table_hbm.at[idx_v.at[b]], rows_v.at[b], sems[b]
        ).start()

    def wait_store(b, c):
        pltpu.make_async_copy(
            table_hbm.at[idx_v.at[b]], rows_v.at[b], sems[b]
        ).wait()
        pltpu.sync_copy(rows_v.at[b], out_hbm.at[pl.ds(base + c * _C, _C)])

    for b in range(_NBUF):
        fire(b, b)

    def loop_body(j, carry):
        for b in range(_NBUF):
            c = j * _NBUF + b
            wait_store(b, c)
            fire(b, c + _NBUF)
        return carry

    n_steady = _CHUNKS // _NBUF - 1
    lax.fori_loop(0, n_steady, loop_body, 0)

    for b in range(_NBUF):
        c = n_steady * _NBUF + b
        wait_store(b, c)


@jax.jit
def _embed_lookup(x_flat, table_padded):
    mesh = plsc.VectorSubcoreMesh(core_axis_name="c", subcore_axis_name="s")
    run = pl.kernel(
        _embed_body,
        mesh=mesh,
        out_type=jax.ShapeDtypeStruct((_N, _DP), jnp.float32),
        scratch_types=[
            pltpu.VMEM((_NBUF, _C), jnp.int32),
            pltpu.VMEM((_NBUF, _C, _DP), jnp.float32),
            pltpu.SemaphoreType.DMA,
            pltpu.SemaphoreType.DMA,
        ],
    )
    return run(x_flat, table_padded)


def kernel(x, table):
    b, h = x.shape
    x_flat = x.reshape(-1).astype(jnp.int32)
    table_padded = jnp.pad(table, ((0, 0), (0, _DP - _D)))
    out = _embed_lookup(x_flat, table_padded)
    return out[:, :_D].reshape(b, h, _D)


# speedup vs baseline: 1.9940x; 1.8181x over previous
import jax
import jax.numpy as jnp
from jax import lax
from jax.experimental import pallas as pl
from jax.experimental.pallas import tpu as pltpu
from jax.experimental.pallas import tpu_sc as plsc

_D = 300
_B = 4096
_H = 50
_NC = 2
_NS = 16
_NW = _NC * _NS
_GPB = 4
_GROUPS_PER_W = _B // (_NW * _GPB)
_CPB = 2


def _embed_body(idx_hbm, table_hbm, out_hbm, idx_v, rows_v, sem):
    wid = lax.axis_index("s") * _NC + lax.axis_index("c")

    def group_body(g, carry):
        gid = wid * _GROUPS_PER_W + g
        pltpu.sync_copy(
            idx_hbm.at[pl.ds(gid * _GPB * _H, _GPB * _H)],
            idx_v.at[pl.ds(0, _GPB * _H)],
        )

        for half in range(2):
            for jb in range(_CPB):
                def fire_row(jr, c, _half=half, _jb=jb):
                    v = idx_v[pl.ds((_half * _CPB + _jb) * _H + jr, 16)]
                    r = v[0]
                    pltpu.make_async_copy(
                        table_hbm.at[r], rows_v.at[_half, _jb, jr], sem
                    ).start()
                    return c
                lax.fori_loop(0, _H, fire_row, 0)

            def drain(k, c, _half=half):
                pltpu.make_async_copy(
                    table_hbm.at[0], rows_v.at[_half, 0, 0], sem
                ).wait()
                return c
            lax.fori_loop(0, _CPB * _H, drain, 0)

            b0 = gid * _GPB + half * _CPB
            pltpu.sync_copy(rows_v.at[half], out_hbm.at[pl.ds(b0, _CPB)])
        return carry

    lax.fori_loop(0, _GROUPS_PER_W, group_body, 0)


@jax.jit
def _embed_lookup(x_flat, table):
    mesh = plsc.VectorSubcoreMesh(core_axis_name="c", subcore_axis_name="s")
    run = pl.kernel(
        _embed_body,
        mesh=mesh,
        out_type=jax.ShapeDtypeStruct((_B, _H, _D), jnp.float32),
        scratch_types=[
            pltpu.VMEM((_GPB * _H + 16,), jnp.int32),
            pltpu.VMEM((2, _CPB, _H, _D), jnp.float32),
            pltpu.SemaphoreType.DMA,
        ],
    )
    return run(x_flat, table)


def kernel(x, table):
    x_flat = x.reshape(-1).astype(jnp.int32)
    return _embed_lookup(x_flat, table)


# speedup vs baseline: 2.1361x; 1.0713x over previous
import jax
import jax.numpy as jnp
from jax import lax
from jax.experimental import pallas as pl
from jax.experimental.pallas import tpu as pltpu
from jax.experimental.pallas import tpu_sc as plsc

_D = 300
_B = 4096
_H = 50
_NC = 2
_NS = 16
_NW = _NC * _NS
_BPW = _B // _NW
_CPB = 2
_CHUNK = _CPB * _H
_NCHUNKS = _BPW // _CPB


def _embed_body(idx_hbm, table_hbm, out_hbm, idx_v, rows_v, g0, g1, s0, s1):
    wid = lax.axis_index("s") * _NC + lax.axis_index("c")
    base = wid * _BPW * _H
    gsems = (g0, g1)
    ssems = (s0, s1)

    pltpu.sync_copy(
        idx_hbm.at[pl.ds(base, _BPW * _H)], idx_v.at[pl.ds(0, _BPW * _H)]
    )

    def out_copy(slot, c):
        b0 = wid * _BPW + c * _CPB
        return pltpu.make_async_copy(
            rows_v.at[slot], out_hbm.at[pl.ds(b0, _CPB)], ssems[slot]
        )

    def fire(slot, c):
        for jb in range(_CPB):
            def fire_row(jr, carry, _jb=jb):
                v = idx_v[pl.ds(c * _CHUNK + _jb * _H + jr, 16)]
                pltpu.make_async_copy(
                    table_hbm.at[v[0]], rows_v.at[slot, _jb, jr], gsems[slot]
                ).start()
                return carry
            lax.fori_loop(0, _H, fire_row, 0, unroll=10)

    def drain_gathers(slot):
        def drain(k, carry):
            pltpu.make_async_copy(
                table_hbm.at[0], rows_v.at[slot, 0, 0], gsems[slot]
            ).wait()
            return carry
        lax.fori_loop(0, _CHUNK, drain, 0, unroll=20)

    for slot in range(2):
        fire(slot, slot)

    def chunk_pair(p, carry):
        c0 = p * 2
        drain_gathers(0)
        out_copy(0, c0).start()
        drain_gathers(1)
        out_copy(1, c0 + 1).start()
        out_copy(0, c0).wait()
        fire(0, c0 + 2)
        out_copy(1, c0 + 1).wait()
        fire(1, c0 + 3)
        return carry

    lax.fori_loop(0, _NCHUNKS // 2 - 1, chunk_pair, 0)

    c0 = _NCHUNKS - 2
    drain_gathers(0)
    out_copy(0, c0).start()
    drain_gathers(1)
    out_copy(1, c0 + 1).start()
    out_copy(0, c0).wait()
    out_copy(1, c0 + 1).wait()


@jax.jit
def _embed_lookup(x_flat, table):
    mesh = plsc.VectorSubcoreMesh(core_axis_name="c", subcore_axis_name="s")
    run = pl.kernel(
        _embed_body,
        mesh=mesh,
        out_type=jax.ShapeDtypeStruct((_B, _H, _D), jnp.float32),
        scratch_types=[
            pltpu.VMEM((_BPW * _H + 16,), jnp.int32),
            pltpu.VMEM((2, _CPB, _H, _D), jnp.float32),
            pltpu.SemaphoreType.DMA,
            pltpu.SemaphoreType.DMA,
            pltpu.SemaphoreType.DMA,
            pltpu.SemaphoreType.DMA,
        ],
    )
    return run(x_flat, table)


def kernel(x, table):
    x_flat = x.reshape(-1).astype(jnp.int32)
    return _embed_lookup(x_flat, table)
